# baseline (speedup 1.0000x reference)
import functools

import jax
import jax.numpy as jnp
from jax import lax
from jax.experimental import pallas as pl
from jax.experimental.pallas import tpu as pltpu
from jax.experimental.pallas import tpu_sc as plsc

_L = 16


def _sc_embed(exp16, table):
    rows, cols = table.shape
    mesh = plsc.VectorSubcoreMesh(core_axis_name="c", subcore_axis_name="s")
    cols_pad = (cols + _L - 1) // _L * _L

    @functools.partial(
        pl.kernel,
        mesh=mesh,
        out_type=jax.ShapeDtypeStruct((cols,), table.dtype),
        scratch_types=[
            pltpu.VMEM((_L,), jnp.int32),
            pltpu.VMEM((rows, cols), jnp.float32),
            pltpu.VMEM((cols_pad,), jnp.float32),
        ],
        compiler_params=pltpu.CompilerParams(needs_layout_passes=False),
    )
    def k(exp_hbm, table_hbm, out_hbm, exp_v, table_v, row_v):
        wid = lax.axis_index("s") * 2 + lax.axis_index("c")

        @pl.when(wid == 0)
        def _():
            pltpu.sync_copy(exp_hbm, exp_v)
            pltpu.sync_copy(table_hbm, table_v)
            row_idx = jnp.clip(exp_v[...], -20, 20) + 20
            lane = lax.iota(jnp.int32, _L)
            for c in range(cols_pad // _L):
                col_idx = jnp.minimum(lane + c * _L, cols - 1)
                row_v[pl.ds(c * _L, _L)] = plsc.load_gather(
                    table_v, [row_idx, col_idx])
            pltpu.sync_copy(row_v.at[pl.ds(0, cols)], out_hbm)

    return k(exp16, table)


def kernel(exponent, E):
    exp16 = jnp.broadcast_to(jnp.asarray(exponent, jnp.int32), (_L,))
    return _sc_embed(exp16, E)


# speedup vs baseline: 1.1035x; 1.1035x over previous
import functools

import jax
import jax.numpy as jnp
from jax import lax
from jax.experimental import pallas as pl
from jax.experimental.pallas import tpu as pltpu
from jax.experimental.pallas import tpu_sc as plsc

_L = 16


def _sc_embed(exp16, table):
    rows, cols = table.shape
    mesh = plsc.VectorSubcoreMesh(
        core_axis_name="c", subcore_axis_name="s", num_cores=1, num_subcores=1)
    cols_pad = (cols + _L - 1) // _L * _L

    @functools.partial(
        pl.kernel,
        mesh=mesh,
        out_type=jax.ShapeDtypeStruct((cols,), table.dtype),
        scratch_types=[
            pltpu.VMEM((_L,), jnp.int32),
            pltpu.VMEM((rows, cols), jnp.float32),
            pltpu.VMEM((cols_pad,), jnp.float32),
            pltpu.SemaphoreType.DMA,
            pltpu.SemaphoreType.DMA,
        ],
        compiler_params=pltpu.CompilerParams(
            needs_layout_passes=False,
            skip_device_barrier=True,
        ),
    )
    def k(exp_hbm, table_hbm, out_hbm, exp_v, table_v, row_v, sem1, sem2):
        cp1 = pltpu.make_async_copy(exp_hbm, exp_v, sem1)
        cp2 = pltpu.make_async_copy(table_hbm, table_v, sem2)
        cp1.start()
        cp2.start()
        cp1.wait()
        row_idx = jnp.clip(exp_v[...], -20, 20) + 20
        lane = lax.iota(jnp.int32, _L)
        cp2.wait()
        for c in range(cols_pad // _L):
            col_idx = jnp.minimum(lane + c * _L, cols - 1)
            row_v[pl.ds(c * _L, _L)] = plsc.load_gather(
                table_v, [row_idx, col_idx])
        pltpu.sync_copy(row_v.at[pl.ds(0, cols)], out_hbm)

    return k(exp16, table)


def kernel(exponent, E):
    exp16 = jnp.broadcast_to(jnp.asarray(exponent, jnp.int32), (_L,))
    return _sc_embed(exp16, E)


# speedup vs baseline: 1.1264x; 1.0207x over previous
import functools

import jax
import jax.numpy as jnp
from jax.experimental import pallas as pl
from jax.experimental.pallas import tpu as pltpu
from jax.experimental.pallas import tpu_sc as plsc

_L = 16


def _sc_embed(exp1, table):
    rows, cols = table.shape
    mesh = plsc.VectorSubcoreMesh(
        core_axis_name="c", subcore_axis_name="s", num_cores=1, num_subcores=1)

    @functools.partial(
        pl.kernel,
        mesh=mesh,
        out_type=jax.ShapeDtypeStruct((cols,), table.dtype),
        scratch_types=[
            pltpu.VMEM((_L,), jnp.int32),
            pltpu.VMEM((_L,), jnp.int32),
            pltpu.VMEM((1, cols), jnp.float32),
            pltpu.SemaphoreType.DMA,
        ],
        compiler_params=pltpu.CompilerParams(
            needs_layout_passes=False,
            use_tc_tiling_on_sc=False,
            skip_device_barrier=True,
        ),
    )
    def k(exp_hbm, table_hbm, out_hbm, exp_v, idx_v, row_v, sem):
        pltpu.sync_copy(exp_hbm, exp_v.at[pl.ds(0, 1)])
        idx_v[...] = jnp.clip(exp_v[...], -20, 20) + 20
        pltpu.async_copy(
            table_hbm.at[idx_v.at[pl.ds(0, 1)]], row_v, sem).wait()
        pltpu.sync_copy(row_v.at[0], out_hbm)

    return k(exp1, table)


def kernel(exponent, E):
    exp1 = jnp.asarray(exponent, jnp.int32).reshape(1)
    return _sc_embed(exp1, E)


# speedup vs baseline: 1.1965x; 1.0623x over previous
import functools

import jax
import jax.numpy as jnp
from jax.experimental import pallas as pl
from jax.experimental.pallas import tpu as pltpu
from jax.experimental.pallas import tpu_sc as plsc


def _sc_embed(exp1, table):
    rows, cols = table.shape
    mesh = plsc.ScalarSubcoreMesh(axis_name="c", num_cores=1)

    @functools.partial(
        pl.kernel,
        mesh=mesh,
        out_type=jax.ShapeDtypeStruct((cols,), table.dtype),
        scratch_types=[
            pltpu.SMEM((1,), jnp.int32),
            pltpu.SemaphoreType.DMA,
        ],
        compiler_params=pltpu.CompilerParams(
            skip_device_barrier=True,
        ),
    )
    def k(exp_hbm, table_hbm, out_hbm, exp_s, sem):
        pltpu.async_copy(exp_hbm, exp_s, sem).wait()
        row = jnp.clip(exp_s[0], -20, 20) + 20
        pltpu.async_copy(table_hbm.at[row], out_hbm, sem).wait()

    return k(exp1, table)


def kernel(exponent, E):
    exp1 = jnp.asarray(exponent, jnp.int32).reshape(1)
    return _sc_embed(exp1, E)
